# baseline (speedup 1.0000x reference)
import functools

import jax
import jax.numpy as jnp
from jax import lax
from jax.experimental import pallas as pl
from jax.experimental.pallas import tpu as pltpu
from jax.experimental.pallas import tpu_sc as plsc


def _gather_kernel(n_l, n_bt, num_cores, embed):
    n_units = n_l * n_bt
    units_per_w = n_units // 32
    n_tr = embed // 8
    mesh = plsc.VectorSubcoreMesh(core_axis_name="c", subcore_axis_name="s")

    scratch = (
        [pltpu.VMEM((units_per_w * 128,), jnp.int32)]
        + [pltpu.VMEM((128, embed), jnp.float32) for _ in range(2)]
        + [pltpu.VMEM((n_tr, 8, 128), jnp.float32) for _ in range(2)]
        + [pltpu.SemaphoreType.DMA for _ in range(4)]
    )

    @functools.partial(
        pl.kernel,
        mesh=mesh,
        out_type=jax.ShapeDtypeStruct((n_l, n_tr, n_bt, 8, 128),
                                      jnp.float32),
        scratch_types=scratch,
        compiler_params=pltpu.CompilerParams(use_tc_tiling_on_sc=False,
                                             needs_layout_passes=False),
    )
    def k(idx_hbm, table_hbm, out_hbm, idx_all, g0, g1, t0, t1,
          gs0, gs1, os0, os1):
        gbuf = (g0, g1)
        tbuf = (t0, t1)
        gsem = (gs0, gs1)
        osem = (os0, os1)
        wid = lax.axis_index("s") * num_cores + lax.axis_index("c")
        ubase = wid * units_per_w

        pltpu.sync_copy(idx_hbm.at[pl.ds(ubase * 128, units_per_w * 128)],
                        idx_all)

        iota = lax.iota(jnp.int32, 16)
        zeros = iota * 0
        tr0 = lax.shift_right_logical(iota, 3)
        s0 = lax.bitwise_and(iota, 7)
        e1 = iota + 16
        tr1 = lax.shift_right_logical(e1, 3)
        s1 = lax.bitwise_and(e1, 7)

        def start_gather(ul, par):
            pltpu.async_copy(
                table_hbm.at[idx_all.at[pl.ds(ul * 128, 128)]],
                gbuf[par], gsem[par])

        def wait_gather(ul, par):
            pltpu.make_async_copy(
                table_hbm.at[idx_all.at[pl.ds(ul * 128, 128)]],
                gbuf[par], gsem[par]).wait()

        def write_out(ul, par):
            u = ubase + ul
            l = u // n_bt
            tc = lax.rem(u, n_bt)
            for tr in range(n_tr):
                pltpu.async_copy(
                    tbuf[par].at[tr], out_hbm.at[l, tr, tc], osem[par])

        def wait_out(ul, par):
            u = ubase + ul
            l = u // n_bt
            tc = lax.rem(u, n_bt)
            for tr in range(n_tr):
                pltpu.make_async_copy(
                    tbuf[par].at[tr], out_hbm.at[l, tr, tc],
                    osem[par]).wait()

        def transpose(par):
            def body(j, carry):
                bj = zeros + j
                r0 = plsc.load_gather(gbuf[par], [bj, iota])
                r1 = plsc.load_gather(gbuf[par], [bj, e1])
                plsc.store_scatter(tbuf[par], [tr0, s0, bj], r0)
                plsc.store_scatter(tbuf[par], [tr1, s1, bj], r1)
                return carry
            lax.fori_loop(0, 128, body, 0, unroll=4)

        def step(ul, par, first, last):
            wait_gather(ul, par)
            if not last:
                start_gather(ul + 1, 1 - par)
            if not first:
                wait_out(ul - 2, par)
            transpose(par)
            write_out(ul, par)

        start_gather(0, 0)
        step(0, 0, True, False)
        step(1, 1, True, False)

        def mid(g, carry):
            ul = 2 * g
            step(ul, 0, False, False)
            step(ul + 1, 1, False, False)
            return carry
        lax.fori_loop(1, units_per_w // 2 - 1, mid, 0)

        step(units_per_w - 2, 0, False, False)
        step(units_per_w - 1, 1, False, True)
        wait_out(units_per_w - 2, 0)
        wait_out(units_per_w - 1, 1)

    return k


def kernel(indices, table):
    b, l = indices.shape
    v, embed = table.shape
    info = plsc.get_sparse_core_info()
    idx_t = indices.T.reshape(b * l)
    x5 = _gather_kernel(l, b // 128, info.num_cores, embed)(idx_t, table)
    return x5.transpose((2, 4, 0, 1, 3)).reshape(b, l, embed)
